# Initial kernel scaffold; baseline (speedup 1.0000x reference)
import functools

import jax
import jax.numpy as jnp
from jax import lax
from jax.experimental import pallas as pl
from jax.experimental.pallas import tpu as pltpu
from jax.experimental.pallas import tpu_sc as plsc

D = 128
LANE = 16
NC, NS = 2, 16
NW = NC * NS


def _prep_body(s_ref, p_ref, x_ref, w_ref, q1_ref, q2_ref, m_ref,
               w1_ref, w2_ref, u_ref, i_ref):
    u_ref[:, :D] = s_ref[...] + p_ref[...]
    u_ref[:, D:] = x_ref[...] * w1_ref[...] + w_ref[...] * w2_ref[...]
    i_ref[:, :D] = q1_ref[...] + q2_ref[...]
    i_ref[:, D:] = m_ref[...]


def _prep_tables(s, p, x, w, q1, q2, m, w1r, w2r):
    n = s.shape[0]
    blk = 2000
    grid = n // blk
    node_spec = pl.BlockSpec((blk, D), lambda i: (i, 0))
    row_spec = pl.BlockSpec((1, D), lambda i: (0, 0))
    out_spec = pl.BlockSpec((blk, 2 * D), lambda i: (i, 0))
    return pl.pallas_call(
        _prep_body,
        grid=(grid,),
        in_specs=[node_spec] * 7 + [row_spec, row_spec],
        out_specs=(out_spec, out_spec),
        out_shape=(jax.ShapeDtypeStruct((n, 2 * D), jnp.float32),
                   jax.ShapeDtypeStruct((n, 2 * D), jnp.float32)),
    )(s, p, x, w, q1, q2, m, w1r, w2r)


def _sc_gather_call(u_tab, i_tab, src, dst, n_edges):
    epw = n_edges // NW
    ch = 80
    nchunk = epw // ch

    mesh = plsc.VectorSubcoreMesh(core_axis_name="c", subcore_axis_name="s")

    @functools.partial(
        pl.kernel,
        out_type=(jax.ShapeDtypeStruct((n_edges, D), jnp.float32),
                  jax.ShapeDtypeStruct((n_edges,), jnp.float32)),
        mesh=mesh,
        scratch_types=[
            pltpu.VMEM((ch,), jnp.int32),
            pltpu.VMEM((ch,), jnp.int32),
            pltpu.VMEM((ch, 2 * D), jnp.float32),
            pltpu.VMEM((ch, 2 * D), jnp.float32),
            pltpu.VMEM((ch, D), jnp.float32),
            pltpu.VMEM((ch,), jnp.float32),
            pltpu.SemaphoreType.DMA,
            pltpu.SemaphoreType.DMA,
        ],
    )
    def sc_kernel(u_hbm, i_hbm, src_hbm, dst_hbm, c_hbm, dot_hbm,
                  src_v, dst_v, u_rows, i_rows, c_st, dot_st, sem_u, sem_i):
        wid = lax.axis_index("s") * NC + lax.axis_index("c")
        base = wid * epw

        def chunk(j, carry):
            off = base + j * ch
            pltpu.sync_copy(src_hbm.at[pl.ds(off, ch)], src_v)
            pltpu.sync_copy(dst_hbm.at[pl.ds(off, ch)], dst_v)
            cp_u = pltpu.async_copy(u_hbm.at[src_v], u_rows, sem_u)
            cp_i = pltpu.async_copy(i_hbm.at[dst_v], i_rows, sem_i)
            cp_u.wait()
            cp_i.wait()

            def edge(k, carry2):
                for r in range(D // LANE):
                    sl = pl.ds(r * LANE, LANE)
                    c_st[k, sl] = u_rows[k, sl] + i_rows[k, sl]
                acc = jnp.zeros((LANE,), jnp.float32)
                for r in range(D // LANE):
                    sl = pl.ds(D + r * LANE, LANE)
                    acc = acc + u_rows[k, sl] * i_rows[k, sl]
                dot_st[k] = jnp.sum(acc)
                return carry2

            lax.fori_loop(0, ch, edge, 0)
            pltpu.sync_copy(c_st, c_hbm.at[pl.ds(off, ch)])
            pltpu.sync_copy(dot_st, dot_hbm.at[pl.ds(off, ch)])
            return carry

        lax.fori_loop(0, nchunk, chunk, 0)

    return sc_kernel(u_tab, i_tab, src, dst)


def _mlp_body(c_ref, dot_ref, wm_ref, bm_ref, wl_ref, bl_ref, b1_ref, b2_ref,
              o_ref):
    c = c_ref[...]
    u = jnp.where(c >= 0, c, 0.01 * c)
    z = lax.dot_general(u, wm_ref[...], (((1,), (1,)), ((), ())),
                        preferred_element_type=jnp.float32)
    z = z + bm_ref[...]
    h = jnp.where(z >= 0, z, 0.01 * z)
    o = jnp.sum(h * wl_ref[...], axis=1, keepdims=True)
    o_ref[...] = o + dot_ref[...] + (bl_ref[...] + b1_ref[...] + b2_ref[...])


def _mlp_call(c, dot_col, W_mlp, bm_row, wl_row, bl, b1, b2):
    n_edges = c.shape[0]
    be = 512
    grid = n_edges // be
    return pl.pallas_call(
        _mlp_body,
        grid=(grid,),
        in_specs=[
            pl.BlockSpec((be, D), lambda i: (i, 0)),
            pl.BlockSpec((be, 1), lambda i: (i, 0)),
            pl.BlockSpec((D, D), lambda i: (0, 0)),
            pl.BlockSpec((1, D), lambda i: (0, 0)),
            pl.BlockSpec((1, D), lambda i: (0, 0)),
            pl.BlockSpec((1, 1), lambda i: (0, 0)),
            pl.BlockSpec((1, 1), lambda i: (0, 0)),
            pl.BlockSpec((1, 1), lambda i: (0, 0)),
        ],
        out_specs=pl.BlockSpec((be, 1), lambda i: (i, 0)),
        out_shape=jax.ShapeDtypeStruct((n_edges, 1), jnp.float32),
    )(c, dot_col, W_mlp, bm_row, wl_row, bl, b1, b2)


def kernel(s, p, x, w, q1, q2, m, edge_index,
           W_mlp, b_mlp, W_L, b_L, W_1, b_1, W_2, b_2):
    n_edges = edge_index.shape[1]
    src = edge_index[0].astype(jnp.int32)
    dst = edge_index[1].astype(jnp.int32)
    w1r = W_1.reshape(1, D)
    w2r = W_2.reshape(1, D)
    u_tab, i_tab = _prep_tables(s, p, x, w, q1, q2, m, w1r, w2r)
    c, dot = _sc_gather_call(u_tab, i_tab, src, dst, n_edges)
    return _mlp_call(c, dot.reshape(n_edges, 1), W_mlp,
                     b_mlp.reshape(1, D), W_L.reshape(1, D),
                     b_L.reshape(1, 1), b_1.reshape(1, 1), b_2.reshape(1, 1))


# speedup vs baseline: 2.5930x; 2.5930x over previous
import functools

import jax
import jax.numpy as jnp
from jax import lax
from jax.experimental import pallas as pl
from jax.experimental.pallas import tpu as pltpu
from jax.experimental.pallas import tpu_sc as plsc

D = 128
LANE = 16
NC, NS = 2, 16
NW = NC * NS


def _prep_body(s_ref, p_ref, x_ref, w_ref, q1_ref, q2_ref, m_ref,
               w1_ref, w2_ref, u_ref, i_ref):
    u_ref[:, :D] = s_ref[...] + p_ref[...]
    u_ref[:, D:] = x_ref[...] * w1_ref[...] + w_ref[...] * w2_ref[...]
    i_ref[:, :D] = q1_ref[...] + q2_ref[...]
    i_ref[:, D:] = m_ref[...]


def _prep_tables(s, p, x, w, q1, q2, m, w1r, w2r):
    n = s.shape[0]
    blk = 2000
    grid = n // blk
    node_spec = pl.BlockSpec((blk, D), lambda i: (i, 0))
    row_spec = pl.BlockSpec((1, D), lambda i: (0, 0))
    out_spec = pl.BlockSpec((blk, 2 * D), lambda i: (i, 0))
    return pl.pallas_call(
        _prep_body,
        grid=(grid,),
        in_specs=[node_spec] * 7 + [row_spec, row_spec],
        out_specs=(out_spec, out_spec),
        out_shape=(jax.ShapeDtypeStruct((n, 2 * D), jnp.float32),
                   jax.ShapeDtypeStruct((n, 2 * D), jnp.float32)),
    )(s, p, x, w, q1, q2, m, w1r, w2r)


def _sc_gather_call(u_tab, i_tab, src, dst, n_edges):
    epw = n_edges // NW
    ch = 80
    nchunk = epw // ch

    mesh = plsc.VectorSubcoreMesh(core_axis_name="c", subcore_axis_name="s")

    @functools.partial(
        pl.kernel,
        out_type=(jax.ShapeDtypeStruct((n_edges, D), jnp.float32),
                  jax.ShapeDtypeStruct((n_edges, LANE), jnp.float32)),
        mesh=mesh,
        scratch_types=[
            pltpu.VMEM((ch,), jnp.int32),
            pltpu.VMEM((ch,), jnp.int32),
            pltpu.VMEM((ch, 2 * D), jnp.float32),
            pltpu.VMEM((ch, 2 * D), jnp.float32),
            pltpu.VMEM((ch, D), jnp.float32),
            pltpu.VMEM((ch, LANE), jnp.float32),
            pltpu.SemaphoreType.DMA,
            pltpu.SemaphoreType.DMA,
        ],
    )
    def sc_kernel(u_hbm, i_hbm, src_hbm, dst_hbm, c_hbm, dot_hbm,
                  src_v, dst_v, u_rows, i_rows, c_st, dot_st, sem_u, sem_i):
        wid = lax.axis_index("s") * NC + lax.axis_index("c")
        base = wid * epw

        def chunk(j, carry):
            off = base + j * ch
            pltpu.sync_copy(src_hbm.at[pl.ds(off, ch)], src_v)
            pltpu.sync_copy(dst_hbm.at[pl.ds(off, ch)], dst_v)
            cp_u = pltpu.async_copy(u_hbm.at[src_v], u_rows, sem_u)
            cp_i = pltpu.async_copy(i_hbm.at[dst_v], i_rows, sem_i)
            cp_u.wait()
            cp_i.wait()

            def edge(k, carry2):
                for r in range(D // LANE):
                    sl = pl.ds(r * LANE, LANE)
                    c_st[k, sl] = u_rows[k, sl] + i_rows[k, sl]
                acc = jnp.zeros((LANE,), jnp.float32)
                for r in range(D // LANE):
                    sl = pl.ds(D + r * LANE, LANE)
                    acc = acc + u_rows[k, sl] * i_rows[k, sl]
                dot_st[k, :] = acc
                return carry2

            lax.fori_loop(0, ch, edge, 0)
            pltpu.sync_copy(c_st, c_hbm.at[pl.ds(off, ch)])
            pltpu.sync_copy(dot_st, dot_hbm.at[pl.ds(off, ch)])
            return carry

        lax.fori_loop(0, nchunk, chunk, 0)

    return sc_kernel(u_tab, i_tab, src, dst)


def _mlp_body(c_ref, dot_ref, wm_ref, bm_ref, wl_ref, bl_ref, b1_ref, b2_ref,
              o_ref):
    c = c_ref[...]
    u = jnp.where(c >= 0, c, 0.01 * c)
    z = lax.dot_general(u, wm_ref[...], (((1,), (1,)), ((), ())),
                        preferred_element_type=jnp.float32)
    z = z + bm_ref[...]
    h = jnp.where(z >= 0, z, 0.01 * z)
    o = jnp.sum(h * wl_ref[...], axis=1, keepdims=True)
    dot = jnp.sum(dot_ref[...], axis=1, keepdims=True)
    o_ref[...] = o + dot + (bl_ref[...] + b1_ref[...] + b2_ref[...])


def _mlp_call(c, dot_col, W_mlp, bm_row, wl_row, bl, b1, b2):
    n_edges = c.shape[0]
    be = 512
    grid = n_edges // be
    return pl.pallas_call(
        _mlp_body,
        grid=(grid,),
        in_specs=[
            pl.BlockSpec((be, D), lambda i: (i, 0)),
            pl.BlockSpec((be, LANE), lambda i: (i, 0)),
            pl.BlockSpec((D, D), lambda i: (0, 0)),
            pl.BlockSpec((1, D), lambda i: (0, 0)),
            pl.BlockSpec((1, D), lambda i: (0, 0)),
            pl.BlockSpec((1, 1), lambda i: (0, 0)),
            pl.BlockSpec((1, 1), lambda i: (0, 0)),
            pl.BlockSpec((1, 1), lambda i: (0, 0)),
        ],
        out_specs=pl.BlockSpec((be, 1), lambda i: (i, 0)),
        out_shape=jax.ShapeDtypeStruct((n_edges, 1), jnp.float32),
    )(c, dot_col, W_mlp, bm_row, wl_row, bl, b1, b2)


def kernel(s, p, x, w, q1, q2, m, edge_index,
           W_mlp, b_mlp, W_L, b_L, W_1, b_1, W_2, b_2):
    n_edges = edge_index.shape[1]
    src = edge_index[0].astype(jnp.int32)
    dst = edge_index[1].astype(jnp.int32)
    w1r = W_1.reshape(1, D)
    w2r = W_2.reshape(1, D)
    u_tab, i_tab = _prep_tables(s, p, x, w, q1, q2, m, w1r, w2r)
    c, dot = _sc_gather_call(u_tab, i_tab, src, dst, n_edges)
    return _mlp_call(c, dot, W_mlp,
                     b_mlp.reshape(1, D), W_L.reshape(1, D),
                     b_L.reshape(1, 1), b_1.reshape(1, 1), b_2.reshape(1, 1))


# speedup vs baseline: 3.6425x; 1.4047x over previous
import functools

import jax
import jax.numpy as jnp
from jax import lax
from jax.experimental import pallas as pl
from jax.experimental.pallas import tpu as pltpu
from jax.experimental.pallas import tpu_sc as plsc

D = 128
LANE = 16
NC, NS = 2, 16
NW = NC * NS


def _prep_body(s_ref, p_ref, x_ref, w_ref, q1_ref, q2_ref, m_ref,
               w1_ref, w2_ref, u_ref, i_ref):
    u_ref[:, :D] = s_ref[...] + p_ref[...]
    u_ref[:, D:] = x_ref[...] * w1_ref[...] + w_ref[...] * w2_ref[...]
    i_ref[:, :D] = q1_ref[...] + q2_ref[...]
    i_ref[:, D:] = m_ref[...]


def _prep_tables(s, p, x, w, q1, q2, m, w1r, w2r):
    n = s.shape[0]
    blk = 2000
    grid = n // blk
    node_spec = pl.BlockSpec((blk, D), lambda i: (i, 0))
    row_spec = pl.BlockSpec((1, D), lambda i: (0, 0))
    out_spec = pl.BlockSpec((blk, 2 * D), lambda i: (i, 0))
    return pl.pallas_call(
        _prep_body,
        grid=(grid,),
        in_specs=[node_spec] * 7 + [row_spec, row_spec],
        out_specs=(out_spec, out_spec),
        out_shape=(jax.ShapeDtypeStruct((n, 2 * D), jnp.float32),
                   jax.ShapeDtypeStruct((n, 2 * D), jnp.float32)),
    )(s, p, x, w, q1, q2, m, w1r, w2r)


def _sc_gather_call(u_tab, i_tab, src, dst, n_edges):
    epw = n_edges // NW
    ch = 40
    nchunk = epw // ch
    npair = nchunk // 2

    mesh = plsc.VectorSubcoreMesh(core_axis_name="c", subcore_axis_name="s")

    @functools.partial(
        pl.kernel,
        out_type=(jax.ShapeDtypeStruct((n_edges, D), jnp.float32),
                  jax.ShapeDtypeStruct((n_edges, LANE), jnp.float32)),
        mesh=mesh,
        scratch_types=[
            pltpu.VMEM((epw,), jnp.int32),
            pltpu.VMEM((epw,), jnp.int32),
            pltpu.VMEM((2, ch, 2 * D), jnp.float32),
            pltpu.VMEM((2, ch, 2 * D), jnp.float32),
            pltpu.VMEM((2, ch, D), jnp.float32),
            pltpu.VMEM((2, ch, LANE), jnp.float32),
            pltpu.SemaphoreType.DMA,
            pltpu.SemaphoreType.DMA,
            pltpu.SemaphoreType.DMA,
            pltpu.SemaphoreType.DMA,
        ],
    )
    def sc_kernel(u_hbm, i_hbm, src_hbm, dst_hbm, c_hbm, dot_hbm,
                  src_v, dst_v, u_rows, i_rows, c_st, dot_st,
                  gsem_a, gsem_b, wsem_a, wsem_b):
        wid = lax.axis_index("s") * NC + lax.axis_index("c")
        base = wid * epw
        gsems = (gsem_a, gsem_b)
        wsems = (wsem_a, wsem_b)

        pltpu.sync_copy(src_hbm.at[pl.ds(base, epw)], src_v)
        pltpu.sync_copy(dst_hbm.at[pl.ds(base, epw)], dst_v)

        def issue_gather(j, b):
            loc = j * ch
            pltpu.async_copy(u_hbm.at[src_v.at[pl.ds(loc, ch)]],
                             u_rows.at[b], gsems[b])
            pltpu.async_copy(i_hbm.at[dst_v.at[pl.ds(loc, ch)]],
                             i_rows.at[b], gsems[b])

        def wait_gather(b):
            pltpu.make_async_copy(u_hbm.at[pl.ds(0, ch)], u_rows.at[b],
                                  gsems[b]).wait()
            pltpu.make_async_copy(i_hbm.at[pl.ds(0, ch)], i_rows.at[b],
                                  gsems[b]).wait()

        def compute(b):
            def edge(k, carry2):
                for r in range(D // LANE):
                    sl = pl.ds(r * LANE, LANE)
                    c_st[b, k, sl] = u_rows[b, k, sl] + i_rows[b, k, sl]
                acc = jnp.zeros((LANE,), jnp.float32)
                for r in range(D // LANE):
                    sl = pl.ds(D + r * LANE, LANE)
                    acc = acc + u_rows[b, k, sl] * i_rows[b, k, sl]
                dot_st[b, k, :] = acc
                return carry2

            lax.fori_loop(0, ch, edge, 0, unroll=2)

        def issue_write(j, b):
            off = base + j * ch
            pltpu.async_copy(c_st.at[b], c_hbm.at[pl.ds(off, ch)], wsems[b])
            pltpu.async_copy(dot_st.at[b], dot_hbm.at[pl.ds(off, ch)],
                             wsems[b])

        def drain_write(b):
            pltpu.make_async_copy(c_st.at[b], c_hbm.at[pl.ds(0, ch)],
                                  wsems[b]).wait()
            pltpu.make_async_copy(dot_st.at[b], dot_hbm.at[pl.ds(0, ch)],
                                  wsems[b]).wait()

        issue_gather(0, 0)
        issue_gather(1, 1)

        def pair(t, carry):
            j = 2 * t

            @pl.when(t >= 1)
            def _():
                drain_write(0)

            wait_gather(0)
            compute(0)
            issue_write(j, 0)

            @pl.when(t < npair - 1)
            def _():
                issue_gather(j + 2, 0)

            @pl.when(t >= 1)
            def _():
                drain_write(1)

            wait_gather(1)
            compute(1)
            issue_write(j + 1, 1)

            @pl.when(t < npair - 1)
            def _():
                issue_gather(j + 3, 1)

            return carry

        lax.fori_loop(0, npair, pair, 0)
        drain_write(0)
        drain_write(1)

    return sc_kernel(u_tab, i_tab, src, dst)


def _mlp_body(c_ref, dot_ref, wm_ref, bm_ref, wl_ref, bl_ref, b1_ref, b2_ref,
              o_ref):
    c = c_ref[...]
    u = jnp.where(c >= 0, c, 0.01 * c)
    z = lax.dot_general(u, wm_ref[...], (((1,), (1,)), ((), ())),
                        preferred_element_type=jnp.float32)
    z = z + bm_ref[...]
    h = jnp.where(z >= 0, z, 0.01 * z)
    o = jnp.sum(h * wl_ref[...], axis=1, keepdims=True)
    dot = jnp.sum(dot_ref[...], axis=1, keepdims=True)
    o_ref[...] = o + dot + (bl_ref[...] + b1_ref[...] + b2_ref[...])


def _mlp_call(c, dot_col, W_mlp, bm_row, wl_row, bl, b1, b2):
    n_edges = c.shape[0]
    be = 512
    grid = n_edges // be
    return pl.pallas_call(
        _mlp_body,
        grid=(grid,),
        in_specs=[
            pl.BlockSpec((be, D), lambda i: (i, 0)),
            pl.BlockSpec((be, LANE), lambda i: (i, 0)),
            pl.BlockSpec((D, D), lambda i: (0, 0)),
            pl.BlockSpec((1, D), lambda i: (0, 0)),
            pl.BlockSpec((1, D), lambda i: (0, 0)),
            pl.BlockSpec((1, 1), lambda i: (0, 0)),
            pl.BlockSpec((1, 1), lambda i: (0, 0)),
            pl.BlockSpec((1, 1), lambda i: (0, 0)),
        ],
        out_specs=pl.BlockSpec((be, 1), lambda i: (i, 0)),
        out_shape=jax.ShapeDtypeStruct((n_edges, 1), jnp.float32),
    )(c, dot_col, W_mlp, bm_row, wl_row, bl, b1, b2)


def kernel(s, p, x, w, q1, q2, m, edge_index,
           W_mlp, b_mlp, W_L, b_L, W_1, b_1, W_2, b_2):
    n_edges = edge_index.shape[1]
    src = edge_index[0].astype(jnp.int32)
    dst = edge_index[1].astype(jnp.int32)
    w1r = W_1.reshape(1, D)
    w2r = W_2.reshape(1, D)
    u_tab, i_tab = _prep_tables(s, p, x, w, q1, q2, m, w1r, w2r)
    c, dot = _sc_gather_call(u_tab, i_tab, src, dst, n_edges)
    return _mlp_call(c, dot, W_mlp,
                     b_mlp.reshape(1, D), W_L.reshape(1, D),
                     b_L.reshape(1, 1), b_1.reshape(1, 1), b_2.reshape(1, 1))
